# Initial kernel scaffold; baseline (speedup 1.0000x reference)
import jax
import jax.numpy as jnp
from jax.experimental import pallas as pl


def kernel(x, edge_index, edge_attr, batch, Wl1, Wr1, We1, att1, b1, Wl2, Wr2, We2, att2, b2, Wl3, Wr3, We3, att3, b3, Wm1, bm1, Wm2, bm2):
    raise NotImplementedError("write your pallas kernel here")


# speedup vs baseline: 1.0025x; 1.0025x over previous
import jax
import jax.numpy as jnp
from jax.experimental import pallas as pl

N = 10000
E = 320000
D_IN = 128
D_EDGE = 16
HID = 16
HEADS = 8
OUT = 128
MLM = 9


def _gatv2(x, edge_index, edge_attr, Wl, Wr, We, att, bias, heads, ch, num_nodes):
    src = edge_index[0]
    dst = edge_index[1]
    x_l = (x @ Wl).reshape(-1, heads, ch)
    x_r = (x @ Wr).reshape(-1, heads, ch)
    e = (edge_attr @ We).reshape(-1, heads, ch)
    x_j = x_l[src]
    x_i = x_r[dst]
    m = x_i + x_j + e
    m = jax.nn.leaky_relu(m, negative_slope=0.2)
    alpha = (m * att[None, :, :]).sum(axis=-1)
    amax = jax.ops.segment_max(alpha, dst, num_segments=num_nodes)
    amax = jnp.where(jnp.isfinite(amax), amax, 0.0)
    ex = jnp.exp(alpha - amax[dst])
    denom = jax.ops.segment_sum(ex, dst, num_segments=num_nodes)
    a = ex / (denom[dst] + 1e-16)
    msg = x_j * a[:, :, None]
    out = jax.ops.segment_sum(msg, dst, num_segments=num_nodes)
    return out.reshape(num_nodes, heads * ch) + bias


def _mlp_body(h_ref, wm1_ref, bm1_ref, wm2_ref, bm2_ref, o_ref):
    h = h_ref[...]
    t = jnp.maximum(h @ wm1_ref[...] + bm1_ref[...][None, :], 0.0)
    o_ref[...] = t @ wm2_ref[...] + bm2_ref[...][None, :]


def _mlp_head(h, Wm1, bm1, Wm2, bm2):
    BN = 2000
    return pl.pallas_call(
        _mlp_body,
        out_shape=jax.ShapeDtypeStruct((N, MLM), jnp.float32),
        grid=(N // BN,),
        in_specs=[
            pl.BlockSpec((BN, OUT), lambda i: (i, 0)),
            pl.BlockSpec((OUT, 128), lambda i: (0, 0)),
            pl.BlockSpec((128,), lambda i: (0,)),
            pl.BlockSpec((128, MLM), lambda i: (0, 0)),
            pl.BlockSpec((MLM,), lambda i: (0,)),
        ],
        out_specs=pl.BlockSpec((BN, MLM), lambda i: (i, 0)),
    )(h, Wm1, bm1, Wm2, bm2)


def kernel(x, edge_index, edge_attr, batch, Wl1, Wr1, We1, att1, b1, Wl2, Wr2, We2, att2, b2, Wl3, Wr3, We3, att3, b3, Wm1, bm1, Wm2, bm2):
    h = _gatv2(x, edge_index, edge_attr, Wl1, Wr1, We1, att1, b1, HEADS, HID, N)
    h = jax.nn.relu(h)
    h = _gatv2(h, edge_index, edge_attr, Wl2, Wr2, We2, att2, b2, HEADS, HID, N)
    h = jax.nn.relu(h)
    h = _gatv2(h, edge_index, edge_attr, Wl3, Wr3, We3, att3, b3, 1, OUT, N)
    return _mlp_head(h, Wm1, bm1, Wm2, bm2)


# speedup vs baseline: 31.3258x; 31.2471x over previous
import functools

import jax
import jax.numpy as jnp
from jax import lax
from jax.experimental import pallas as pl
from jax.experimental.pallas import tpu as pltpu
from jax.experimental.pallas import tpu_sc as plsc

N = 10000
E = 320000
D_IN = 128
D_EDGE = 16
HID = 16
HEADS = 8
OUT = 128
MLM = 9

NC = 2
NS = 16
NTILES = NC * NS
EPT = E // NTILES
CHUNK = 80
NCHUNK = EPT // CHUNK
NPAD = 10240
ROWS_PER_TILE = NPAD // NS
DEN_R = NPAD // 8


def _sc_edge_pass(heads):
    mesh = plsc.VectorSubcoreMesh(
        core_axis_name="c", subcore_axis_name="s", num_cores=NC, num_subcores=NS)

    @functools.partial(
        pl.kernel,
        out_type=(
            jax.ShapeDtypeStruct((NC, NPAD, 128), jnp.float32),
            jax.ShapeDtypeStruct((NC, DEN_R, 128), jnp.float32),
        ),
        mesh=mesh,
        scratch_types=[
            pltpu.VMEM((CHUNK,), jnp.int32),
            pltpu.VMEM((CHUNK,), jnp.int32),
            pltpu.VMEM((CHUNK,), jnp.int32),
            pltpu.VMEM((CHUNK, 128), jnp.float32),
            pltpu.VMEM((CHUNK, 128), jnp.float32),
            pltpu.VMEM((CHUNK, 128), jnp.float32),
            pltpu.VMEM((128,), jnp.float32),
            pltpu.VMEM_SHARED((NPAD, 128), jnp.float32),
            pltpu.VMEM_SHARED((DEN_R, 128), jnp.float32),
            pltpu.SemaphoreType.DMA,
            pltpu.SemaphoreType.DMA,
            pltpu.SemaphoreType.DMA,
        ],
    )
    def body(src_hbm, dst_hbm, xl_hbm, xr_hbm, e_hbm, att_hbm, zeros_hbm,
             out_hbm, outden_hbm, src_v, dst_v, q2_v, xj_v, xi_v, e_v,
             att_v, acc_sh, den_sh, sem1, sem2, sem3):
        cid = lax.axis_index("c")
        sid = lax.axis_index("s")
        tid = cid * NS + sid

        pltpu.sync_copy(zeros_hbm, acc_sh.at[pl.ds(sid * ROWS_PER_TILE, ROWS_PER_TILE)])
        pltpu.sync_copy(zeros_hbm.at[pl.ds(0, DEN_R // NS)],
                        den_sh.at[pl.ds(sid * (DEN_R // NS), DEN_R // NS)])
        pltpu.sync_copy(att_hbm, att_v)
        plsc.subcore_barrier()

        lane = lax.iota(jnp.int32, 16)
        lane_half = lax.shift_right_logical(lane, 3)
        lane7 = lane & 7
        att_regs = [att_v[pl.ds(16 * h, 16)] for h in range(8)]

        def splat_sum(v):
            for k in (8, 4, 2, 1):
                v = v + v.at[lane ^ k].get(mode="promise_in_bounds")
            return v

        zero16 = jnp.zeros((16,), jnp.float32)

        def den_store(i, dstg, j, ex_row):
            dsp = dstg.at[jnp.full((16,), j, jnp.int32)].get(mode="promise_in_bounds")
            ex16 = ex_row.at[lane7].get(mode="promise_in_bounds")
            dq = dsp & 7
            for w in range(4):
                xi_v[i, pl.ds(16 * w, 16)] = jnp.where(
                    lane_half + 2 * w == dq, ex16, zero16)

        def group_body8(g, carry):
            dstg = dst_v[pl.ds(g * 16, 16)]
            q2_v[pl.ds(g * 16, 16)] = lax.shift_right_logical(dstg, 3)
            for j in range(16):
                i = g * 16 + j
                ex_row = jnp.zeros((16,), jnp.float32)
                for h in range(8):
                    sl = pl.ds(h * 16, 16)
                    xj = xj_v[i, sl]
                    m = xi_v[i, sl] + xj + e_v[i, sl]
                    m = jnp.maximum(m, m * 0.2)
                    exh = jnp.exp(splat_sum(m * att_regs[h]))
                    xj_v[i, sl] = xj * exh
                    ex_row = jnp.where(lane == h, exh, ex_row)
                den_store(i, dstg, j, ex_row)
            return carry

        def group_body1(g, carry):
            dstg = dst_v[pl.ds(g * 16, 16)]
            q2_v[pl.ds(g * 16, 16)] = lax.shift_right_logical(dstg, 3)
            for j in range(16):
                i = g * 16 + j
                acc = jnp.zeros((16,), jnp.float32)
                xjs = []
                for h in range(8):
                    sl = pl.ds(h * 16, 16)
                    xj = xj_v[i, sl]
                    xjs.append(xj)
                    m = xi_v[i, sl] + xj + e_v[i, sl]
                    m = jnp.maximum(m, m * 0.2)
                    acc = acc + m * att_regs[h]
                exh = jnp.exp(splat_sum(acc))
                for h in range(8):
                    xj_v[i, pl.ds(h * 16, 16)] = xjs[h] * exh
                ex_row = jnp.where(lane == 0, exh, zero16)
                den_store(i, dstg, j, ex_row)
            return carry

        group_body = group_body8 if heads == 8 else group_body1

        def chunk_body(k, carry):
            base = tid * EPT + k * CHUNK
            pltpu.sync_copy(src_hbm.at[pl.ds(base, CHUNK)], src_v)
            pltpu.sync_copy(dst_hbm.at[pl.ds(base, CHUNK)], dst_v)
            cp1 = pltpu.async_copy(xl_hbm.at[src_v], xj_v, sem1)
            cp2 = pltpu.async_copy(xr_hbm.at[dst_v], xi_v, sem2)
            cp3 = pltpu.async_copy(e_hbm.at[pl.ds(base, CHUNK)], e_v, sem3)
            cp1.wait()
            cp2.wait()
            cp3.wait()
            lax.fori_loop(0, CHUNK // 16, group_body, 0)
            pltpu.sync_copy(xj_v, acc_sh.at[dst_v], add=True)
            pltpu.sync_copy(xi_v, den_sh.at[q2_v], add=True)
            return carry

        lax.fori_loop(0, NCHUNK, chunk_body, 0)

        plsc.subcore_barrier()
        pltpu.sync_copy(
            acc_sh.at[pl.ds(sid * ROWS_PER_TILE, ROWS_PER_TILE)],
            out_hbm.at[cid, pl.ds(sid * ROWS_PER_TILE, ROWS_PER_TILE)])
        pltpu.sync_copy(
            den_sh.at[pl.ds(sid * (DEN_R // NS), DEN_R // NS)],
            outden_hbm.at[cid, pl.ds(sid * (DEN_R // NS), DEN_R // NS)])

    return body


_sc_edge8 = _sc_edge_pass(8)
_sc_edge1 = _sc_edge_pass(1)


def _edge_proj_body(ea_ref, w1_ref, w2_ref, w3_ref, o1_ref, o2_ref, o3_ref):
    ea = ea_ref[...]
    o1_ref[...] = ea @ w1_ref[...]
    o2_ref[...] = ea @ w2_ref[...]
    o3_ref[...] = ea @ w3_ref[...]


def _edge_proj(edge_attr, We1, We2, We3):
    BE = 8000
    sh = jax.ShapeDtypeStruct((E, 128), jnp.float32)
    return pl.pallas_call(
        _edge_proj_body,
        out_shape=(sh, sh, sh),
        grid=(E // BE,),
        in_specs=[
            pl.BlockSpec((BE, D_EDGE), lambda i: (i, 0)),
            pl.BlockSpec((D_EDGE, 128), lambda i: (0, 0)),
            pl.BlockSpec((D_EDGE, 128), lambda i: (0, 0)),
            pl.BlockSpec((D_EDGE, 128), lambda i: (0, 0)),
        ],
        out_specs=(
            pl.BlockSpec((BE, 128), lambda i: (i, 0)),
            pl.BlockSpec((BE, 128), lambda i: (i, 0)),
            pl.BlockSpec((BE, 128), lambda i: (i, 0)),
        ),
    )(edge_attr, We1, We2, We3)


def _node_proj_body(x_ref, wl_ref, wr_ref, xl_ref, xr_ref):
    x = x_ref[...]
    xl_ref[...] = x @ wl_ref[...]
    xr_ref[...] = x @ wr_ref[...]


def _node_proj1(x, Wl, Wr):
    BN = 2000
    sh = jax.ShapeDtypeStruct((N, 128), jnp.float32)
    return pl.pallas_call(
        _node_proj_body,
        out_shape=(sh, sh),
        grid=(N // BN,),
        in_specs=[
            pl.BlockSpec((BN, 128), lambda i: (i, 0)),
            pl.BlockSpec((128, 128), lambda i: (0, 0)),
            pl.BlockSpec((128, 128), lambda i: (0, 0)),
        ],
        out_specs=(
            pl.BlockSpec((BN, 128), lambda i: (i, 0)),
            pl.BlockSpec((BN, 128), lambda i: (i, 0)),
        ),
    )(x, Wl, Wr)


def _combine(acc, den, bias, Wl, Wr, ch):
    BN = 2000

    def body(acc_ref, den_ref, b_ref, wl_ref, wr_ref, xl_ref, xr_ref):
        msg = acc_ref[0] + acc_ref[1]
        den_blk = den_ref[0] + den_ref[1]
        r = (lax.broadcasted_iota(jnp.int32, (8, 128), 1) // ch
             == lax.broadcasted_iota(jnp.int32, (8, 128), 0)).astype(jnp.float32)
        denb = den_blk @ r
        h = msg / (denb + 1e-16) + b_ref[...][None, :]
        h = jnp.maximum(h, 0.0)
        xl_ref[...] = h @ wl_ref[...]
        xr_ref[...] = h @ wr_ref[...]

    sh = jax.ShapeDtypeStruct((N, 128), jnp.float32)
    return pl.pallas_call(
        body,
        out_shape=(sh, sh),
        grid=(N // BN,),
        in_specs=[
            pl.BlockSpec((NC, BN, 128), lambda i: (0, i, 0)),
            pl.BlockSpec((NC, BN, 8), lambda i: (0, i, 0)),
            pl.BlockSpec((128,), lambda i: (0,)),
            pl.BlockSpec((128, 128), lambda i: (0, 0)),
            pl.BlockSpec((128, 128), lambda i: (0, 0)),
        ],
        out_specs=(
            pl.BlockSpec((BN, 128), lambda i: (i, 0)),
            pl.BlockSpec((BN, 128), lambda i: (i, 0)),
        ),
    )(acc, den, bias, Wl, Wr)


def _final(acc, den, b3, Wm1, bm1, Wm2, bm2):
    BN = 2000

    def body(acc_ref, den_ref, b_ref, wm1_ref, bm1_ref, wm2_ref, bm2_ref, o_ref):
        msg = acc_ref[0] + acc_ref[1]
        den_blk = den_ref[0] + den_ref[1]
        r = (lax.broadcasted_iota(jnp.int32, (8, 128), 0) == 0).astype(jnp.float32)
        denb = den_blk @ r
        h = msg / (denb + 1e-16) + b_ref[...][None, :]
        t = jnp.maximum(h @ wm1_ref[...] + bm1_ref[...][None, :], 0.0)
        o_ref[...] = t @ wm2_ref[...] + bm2_ref[...][None, :]

    return pl.pallas_call(
        body,
        out_shape=jax.ShapeDtypeStruct((N, MLM), jnp.float32),
        grid=(N // BN,),
        in_specs=[
            pl.BlockSpec((NC, BN, 128), lambda i: (0, i, 0)),
            pl.BlockSpec((NC, BN, 8), lambda i: (0, i, 0)),
            pl.BlockSpec((128,), lambda i: (0,)),
            pl.BlockSpec((128, 128), lambda i: (0, 0)),
            pl.BlockSpec((128,), lambda i: (0,)),
            pl.BlockSpec((128, MLM), lambda i: (0, 0)),
            pl.BlockSpec((MLM,), lambda i: (0,)),
        ],
        out_specs=pl.BlockSpec((BN, MLM), lambda i: (i, 0)),
    )(acc, den, b3, Wm1, bm1, Wm2, bm2)


def kernel(x, edge_index, edge_attr, batch, Wl1, Wr1, We1, att1, b1, Wl2, Wr2, We2, att2, b2, Wl3, Wr3, We3, att3, b3, Wm1, bm1, Wm2, bm2):
    src = edge_index[0].astype(jnp.int32)
    dst = edge_index[1].astype(jnp.int32)
    zeros = jnp.zeros((ROWS_PER_TILE, 128), jnp.float32)
    att1f = att1.reshape(-1)
    att2f = att2.reshape(-1)
    att3f = att3.reshape(-1)

    def den_view(den):
        return den.reshape(NC, DEN_R, 16, 8)[:, :, :8, :].reshape(NC, NPAD, 8)

    e1, e2, e3 = _edge_proj(edge_attr, We1, We2, We3)
    xl1, xr1 = _node_proj1(x, Wl1, Wr1)
    acc1, den1 = _sc_edge8(src, dst, xl1, xr1, e1, att1f, zeros)
    xl2, xr2 = _combine(acc1, den_view(den1), b1, Wl2, Wr2, HID)
    acc2, den2 = _sc_edge8(src, dst, xl2, xr2, e2, att2f, zeros)
    xl3, xr3 = _combine(acc2, den_view(den2), b2, Wl3, Wr3, HID)
    acc3, den3 = _sc_edge1(src, dst, xl3, xr3, e3, att3f, zeros)
    return _final(acc3, den_view(den3), b3, Wm1, bm1, Wm2, bm2)
